# baseline (speedup 1.0000x reference)
import jax
import jax.numpy as jnp
from jax import lax
from jax.experimental import pallas as pl
from jax.experimental.pallas import tpu as pltpu
from jax.experimental.pallas import tpu_sc as plsc

B = 32
T = 2048
D = 128
C = 128
NC = 2
NS = 16
NW = NC * NS
NTC = T // C
NBG = NW // NTC
GB = B // NBG
LANES = 16
NBUF = 6
DEPTH = 5


def _emb_body(x_hbm, tok_hbm, pos_hbm, out_hbm, idx_v, spos, *rest):
    toks = rest[:NBUF]
    psem = rest[NBUF]
    isem = rest[NBUF + 1]
    gsems = rest[NBUF + 2:2 * NBUF + 2]
    osems = rest[2 * NBUF + 2:3 * NBUF + 2]
    fsems = rest[3 * NBUF + 2:]

    wid = lax.axis_index("s") * NC + lax.axis_index("c")
    tc = wid // NBG
    bg = wid % NBG

    def row_base(g):
        return (bg * GB + g) * T + tc * C

    pos_rows = pl.ds(tc * C, C)
    idx_copy = pltpu.async_copy(
        x_hbm.at[pl.ds(bg * GB, GB), pl.ds(tc * C, C)], idx_v, isem)
    pos_copy = pltpu.async_copy(pos_hbm.at[pos_rows], spos.at[pos_rows], psem)

    prefills = [None] * GB
    gathers = [None] * GB
    out_copies = [None] * NBUF

    for p in range(NBUF):
        prefills[p] = pltpu.async_copy(pos_hbm.at[pos_rows], toks[p], fsems[p])

    def issue_prefill(g):
        buf = g % NBUF
        if out_copies[buf] is not None:
            out_copies[buf].wait()
            out_copies[buf] = None
        prefills[g] = pltpu.async_copy(spos.at[pos_rows], toks[buf], fsems[buf])

    def issue_gather(g):
        buf = g % NBUF
        prefills[g].wait()
        gathers[g] = pltpu.async_copy(
            tok_hbm.at[idx_v.at[g]], toks[buf], gsems[buf], add=True)

    idx_copy.wait()
    for p in range(DEPTH):
        issue_gather(p)
    pos_copy.wait()
    for g in range(GB):
        buf = g % NBUF
        gathers[g].wait()
        out_copies[buf] = pltpu.async_copy(
            toks[buf], out_hbm.at[pl.ds(row_base(g), C)], osems[buf])
        if g + NBUF < GB:
            issue_prefill(g + NBUF)
        if g + DEPTH < GB:
            issue_gather(g + DEPTH)

    for oc in out_copies:
        if oc is not None:
            oc.wait()


@jax.jit
def _emb_call(x2d, token_table, pos_table):
    mesh = plsc.VectorSubcoreMesh(
        core_axis_name="c", subcore_axis_name="s", num_cores=NC, num_subcores=NS
    )
    f = pl.kernel(
        _emb_body,
        out_type=jax.ShapeDtypeStruct((B * T, D), jnp.float32),
        mesh=mesh,
        scratch_types=[
            pltpu.VMEM((GB, C), jnp.int32),
            pltpu.VMEM_SHARED((T, D), jnp.float32),
        ] + [pltpu.VMEM((C, D), jnp.float32) for _ in range(NBUF)]
          + [pltpu.SemaphoreType.DMA for _ in range(2 + 3 * NBUF)],
    )
    return f(x2d, token_table, pos_table)


def kernel(x, token_table, pos_table):
    out = _emb_call(x.astype(jnp.int32), token_table, pos_table)
    return out.reshape(B, T, D)


# speedup vs baseline: 1.1192x; 1.1192x over previous
import jax
import jax.numpy as jnp
from jax import lax
from jax.experimental import pallas as pl
from jax.experimental.pallas import tpu as pltpu
from jax.experimental.pallas import tpu_sc as plsc

B = 32
T = 2048
D = 128
C = 128
NC = 2
NS = 16
NW = NC * NS
NTC = T // C
NBG = NW // NTC
GB = B // NBG
LANES = 16
NBUF = 6
DEPTH = 5


def _emb_body(x_hbm, tok_hbm, pos_hbm, out_hbm, pos_v, idx_v, spos, *rest):
    toks = rest[:NBUF]
    psem = rest[NBUF]
    isem = rest[NBUF + 1]
    gsems = rest[NBUF + 2:2 * NBUF + 2]
    osems = rest[2 * NBUF + 2:3 * NBUF + 2]
    fsems = rest[3 * NBUF + 2:]

    wid = lax.axis_index("s") * NC + lax.axis_index("c")
    tc = wid // NBG
    bg = wid % NBG

    def row_base(g):
        return (bg * GB + g) * T + tc * C

    pos_copy = pltpu.async_copy(
        pos_hbm.at[pl.ds(tc * C, C)], spos.at[pl.ds(tc * C, C)], psem)
    pltpu.sync_copy(
        x_hbm.at[pl.ds(bg * GB, GB), pl.ds(tc * C, C)], idx_v)

    def add_pos(tok_ref):
        @plsc.parallel_loop(0, C, step=1, unroll=4)
        def row_body(r):
            for j in range(D // LANES):
                sl = pl.ds(j * LANES, LANES)
                plsc.addupdate(tok_ref.at[r, sl], pos_v[r, sl])

    gathers = [None] * GB
    out_copies = [None] * NBUF

    prefills = [None] * GB

    def issue_prefill(g):
        buf = g % NBUF
        if out_copies[buf] is not None:
            out_copies[buf].wait()
            out_copies[buf] = None
        prefills[g] = pltpu.async_copy(
            spos.at[pl.ds(tc * C, C)], toks[buf], fsems[buf])

    def issue_gather(g):
        buf = g % NBUF
        prefills[g].wait()
        gathers[g] = pltpu.async_copy(
            tok_hbm.at[idx_v.at[g]], toks[buf], gsems[buf], add=True)

    pos_copy.wait()
    for p in range(DEPTH + 1):
        issue_prefill(p)
    for p in range(DEPTH):
        issue_gather(p)
    for g in range(GB):
        buf = g % NBUF
        gathers[g].wait()
        out_copies[buf] = pltpu.async_copy(
            toks[buf], out_hbm.at[pl.ds(row_base(g), C)], osems[buf])
        if g + DEPTH + 1 < GB:
            issue_prefill(g + DEPTH + 1)
        if g + DEPTH < GB:
            issue_gather(g + DEPTH)

    for oc in out_copies:
        if oc is not None:
            oc.wait()


@jax.jit
def _emb_call(x2d, token_table, pos_table):
    mesh = plsc.VectorSubcoreMesh(
        core_axis_name="c", subcore_axis_name="s", num_cores=NC, num_subcores=NS
    )
    f = pl.kernel(
        _emb_body,
        out_type=jax.ShapeDtypeStruct((B * T, D), jnp.float32),
        mesh=mesh,
        scratch_types=[
            pltpu.VMEM((C, D), jnp.float32),
            pltpu.VMEM((GB, C), jnp.int32),
            pltpu.VMEM_SHARED((T, D), jnp.float32),
        ] + [pltpu.VMEM((C, D), jnp.float32) for _ in range(NBUF)]
          + [pltpu.SemaphoreType.DMA for _ in range(2 + 3 * NBUF)],
    )
    return f(x2d, token_table, pos_table)


def kernel(x, token_table, pos_table):
    out = _emb_call(x.astype(jnp.int32), token_table, pos_table)
    return out.reshape(B, T, D)
